# baseline (speedup 1.0000x reference)
import functools

import jax
import jax.numpy as jnp
from jax import lax
from jax.experimental import pallas as pl
from jax.experimental.pallas import tpu as pltpu
from jax.experimental.pallas import tpu_sc as plsc

GAMMA = 2
LAMB = max(5.0, 1500.0 / 1.001)
INV1PL = 1.0 / (1.0 + LAMB)


def _loss_body(cos_ref, tgt_ref, pt_ref, out_ref, *, n_rows, n_cols):
    i = pl.program_id(0)
    cos = cos_ref[...]
    tgt = tgt_ref[0, 0, :]
    p_t = pt_ref[0, 0, :]
    col = lax.broadcasted_iota(jnp.int32, (n_rows, n_cols), 1)
    mask = col == tgt[:, None]
    c_t = jnp.sum(jnp.where(mask, cos, 0.0), axis=1)
    mod = c_t + (p_t - c_t) * INV1PL
    m = jnp.maximum(jnp.max(jnp.where(mask, -jnp.inf, cos), axis=1), mod)
    s = (jnp.sum(jnp.where(mask, 0.0, jnp.exp(cos - m[:, None])), axis=1)
         + jnp.exp(mod - m))
    logpt = mod - m - jnp.log(s)
    pt = jnp.exp(logpt)
    blk = jnp.sum(-((1.0 - pt) ** GAMMA) * logpt)

    @pl.when(i == 0)
    def _():
        out_ref[0, 0] = 0.0

    out_ref[0, 0] += blk


def _make_loss_call(B, C, BR):
    nblk = B // BR
    body = functools.partial(_loss_body, n_rows=BR, n_cols=C)
    return pl.pallas_call(
        body,
        grid=(nblk,),
        in_specs=[
            pl.BlockSpec((BR, C), lambda i: (i, 0)),
            pl.BlockSpec((1, 1, BR), lambda i: (i, 0, 0)),
            pl.BlockSpec((1, 1, BR), lambda i: (i, 0, 0)),
        ],
        out_specs=pl.BlockSpec(memory_space=pltpu.SMEM),
        out_shape=jax.ShapeDtypeStruct((1, 1), jnp.float32),
    )


def _make_sc_gather(B, C):
    info = plsc.get_sparse_core_info()
    nc, ns, lanes = info.num_cores, info.num_subcores, info.num_lanes
    nw = nc * ns
    per_w = B // nw

    mesh = plsc.VectorSubcoreMesh(core_axis_name="c", subcore_axis_name="s")

    @functools.partial(
        pl.kernel,
        mesh=mesh,
        out_type=jax.ShapeDtypeStruct((B,), jnp.float32),
        scratch_types=[
            pltpu.VMEM((per_w,), jnp.int32),
            pltpu.VMEM((per_w,), jnp.int32),
            pltpu.VMEM((per_w,), jnp.float32),
            pltpu.SemaphoreType.DMA,
        ],
    )
    def gather(phi_hbm, tgt_hbm, out_hbm, tgt_v, idx_v, val_v, sem):
        wid = lax.axis_index("s") * nc + lax.axis_index("c")
        base = wid * per_w
        pltpu.sync_copy(tgt_hbm.at[pl.ds(base, per_w)], tgt_v)
        for j in range(per_w // lanes):
            row = base + j * lanes + lax.broadcasted_iota(jnp.int32, (lanes,), 0)
            idx_v[pl.ds(j * lanes, lanes)] = tgt_v[pl.ds(j * lanes, lanes)] + row * C
        pltpu.async_copy(phi_hbm.at[idx_v], val_v, sem).wait()
        pltpu.sync_copy(val_v, out_hbm.at[pl.ds(base, per_w)])

    return gather


def kernel(cos_theta, phi_theta, xlen, target):
    del xlen
    B, C = cos_theta.shape
    p_t = _make_sc_gather(B, C)(jnp.reshape(phi_theta, (-1,)), target)
    BR = 256
    nblk = B // BR
    tgt3 = jnp.reshape(target, (nblk, 1, BR))
    pt3 = jnp.reshape(p_t, (nblk, 1, BR))
    total = _make_loss_call(B, C, BR)(cos_theta, tgt3, pt3)
    return total[0, 0] / B
